# speedup vs baseline: 1.2149x; 1.2149x over previous
import functools

import jax
import jax.numpy as jnp
from jax import lax
from jax.experimental import pallas as pl


def _proj_body(f_ref, w_ref, o_ref):
    o_ref[...] = jax.lax.dot_general(
        f_ref[...], w_ref[...], (((1,), (0,)), ((), ())),
        preferred_element_type=jnp.float32,
        precision=jax.lax.Precision.HIGHEST,
    )


def _combine_body(pmax_ref, pmin_ref, scale_ref, shift_ref, o_ref):
    scale = scale_ref[...]
    shift = shift_ref[...]
    g = jnp.where(scale >= 0.0, pmax_ref[...], pmin_ref[...])
    o_ref[...] = jnp.maximum(g * scale + shift, 0.0)


def kernel(xyz, feature, npoint, W, gamma, beta):
    del npoint
    B, N, Cin = feature.shape
    Cout = W.shape[1]
    K = 16

    sq = jnp.sum(xyz * xyz, axis=-1)
    dist = -2.0 * jnp.matmul(xyz, jnp.transpose(xyz, (0, 2, 1)))
    dist = dist + sq[:, :, None] + sq[:, None, :]
    _, idx = jax.lax.top_k(-dist, K)

    RB = 1024
    proj = pl.pallas_call(
        _proj_body,
        grid=(B * N // RB,),
        in_specs=[
            pl.BlockSpec((RB, Cin), lambda i: (i, 0)),
            pl.BlockSpec((Cin, Cout), lambda i: (0, 0)),
        ],
        out_specs=pl.BlockSpec((RB, Cout), lambda i: (i, 0)),
        out_shape=jax.ShapeDtypeStruct((B * N, Cout), jnp.float32),
    )(feature.reshape(B * N, Cin), W)

    flat_idx = (idx + (jnp.arange(B, dtype=idx.dtype) * N)[:, None, None]).reshape(-1)
    counts = jnp.zeros((B * N,), jnp.float32).at[flat_idx].add(1.0)
    M = B * N * K
    s1 = jnp.sum(counts[:, None] * proj, axis=0)
    s2 = jnp.sum(counts[:, None] * (proj * proj), axis=0)
    mean = s1 / M
    var = s2 / M - mean * mean
    scale = gamma * jax.lax.rsqrt(var + 1e-5)
    shift = beta - mean * scale

    g = proj[flat_idx].reshape(B, N, K, Cout)
    pmax = jnp.max(g, axis=2).reshape(B * N, Cout)
    pmin = jnp.min(g, axis=2).reshape(B * N, Cout)

    out = pl.pallas_call(
        _combine_body,
        grid=(B * N // RB,),
        in_specs=[
            pl.BlockSpec((RB, Cout), lambda i: (i, 0)),
            pl.BlockSpec((RB, Cout), lambda i: (i, 0)),
            pl.BlockSpec((1, Cout), lambda i: (0, 0)),
            pl.BlockSpec((1, Cout), lambda i: (0, 0)),
        ],
        out_specs=pl.BlockSpec((RB, Cout), lambda i: (i, 0)),
        out_shape=jax.ShapeDtypeStruct((B * N, Cout), jnp.float32),
    )(pmax, pmin, scale.reshape(1, Cout), shift.reshape(1, Cout))

    return (xyz, out.reshape(B, N, Cout))


# speedup vs baseline: 1.4963x; 1.2317x over previous
import functools

import jax
import jax.numpy as jnp
from jax import lax
from jax.experimental import pallas as pl

_NEG = 3.0e38


def _proj_body(f_ref, w_ref, o_ref):
    o_ref[...] = jax.lax.dot_general(
        f_ref[...], w_ref[...], (((1,), (0,)), ((), ())),
        preferred_element_type=jnp.float32,
        precision=jax.lax.Precision.HIGHEST,
    )


def _combine_body(pmax_ref, pmin_ref, scale_ref, shift_ref, o_ref):
    scale = scale_ref[...]
    shift = shift_ref[...]
    g = jnp.where(scale >= 0.0, pmax_ref[...], pmin_ref[...])
    o_ref[...] = jnp.maximum(g * scale + shift, 0.0)


def _bf(x):
    return x.astype(jnp.bfloat16).astype(jnp.float32)


def _dist_like_ref(rows, cands):
    s = _bf(rows[0]) * _bf(cands[0])
    s = s + _bf(rows[1]) * _bf(cands[1])
    s = s + _bf(rows[2]) * _bf(cands[2])
    nr = rows[0] * rows[0] + rows[1] * rows[1] + rows[2] * rows[2]
    nc = cands[0] * cands[0] + cands[1] * cands[1] + cands[2] * cands[2]
    return (-2.0 * s + nr) + nc


def _cmin_body(xyz_ref, xyzt_ref, cmin_ref):
    rb = xyz_ref.shape[0]
    n = xyzt_ref.shape[2]
    rows = [xyz_ref[:, d:d + 1] for d in range(3)]
    cands = [xyzt_ref[0, d:d + 1, :] for d in range(3)]
    dist = _dist_like_ref(rows, cands)
    cmin_ref[...] = jnp.min(dist.reshape(rb, n // 16, 16), axis=-1)


def _stage1_body(cmin_ref, out_ref):
    rb, nc = cmin_ref.shape
    v = cmin_ref[...]
    iota = lax.broadcasted_iota(jnp.int32, (rb, nc), 1).astype(jnp.float32)
    for t in range(16):
        m = jnp.min(v, axis=1, keepdims=True)
        eq = v <= m
        pos = jnp.min(jnp.where(eq, iota, _NEG), axis=1, keepdims=True)
        out_ref[:, t:t + 1] = pos.astype(jnp.int32)
        v = jnp.where(iota == pos, _NEG, v)


def _stage2_body(gx_ref, gy_ref, gz_ref, gidx_ref, xyz_ref, out_ref):
    rb, nc = gx_ref.shape
    rows = [xyz_ref[:, d:d + 1] for d in range(3)]
    v = _dist_like_ref(rows, [gx_ref[...], gy_ref[...], gz_ref[...]])
    gidx = gidx_ref[...]
    for t in range(16):
        m = jnp.min(v, axis=1, keepdims=True)
        eq = v <= m
        idx = jnp.min(jnp.where(eq, gidx, _NEG), axis=1, keepdims=True)
        out_ref[:, t:t + 1] = idx.astype(jnp.int32)
        v = jnp.where(jnp.logical_and(eq, gidx == idx), _NEG, v)


def kernel(xyz, feature, npoint, W, gamma, beta):
    del npoint
    B, N, Cin = feature.shape
    Cout = W.shape[1]
    K = 16
    NCH = N // 16

    RB = 256
    xyz_flat = xyz.reshape(B * N, 3)
    xyzt = jnp.transpose(xyz, (0, 2, 1))
    cmin = pl.pallas_call(
        _cmin_body,
        grid=(B, N // RB),
        in_specs=[
            pl.BlockSpec((RB, 3), lambda b, i: (b * (N // RB) + i, 0)),
            pl.BlockSpec((1, 3, N), lambda b, i: (b, 0, 0)),
        ],
        out_specs=pl.BlockSpec((RB, NCH), lambda b, i: (b * (N // RB) + i, 0)),
        out_shape=jax.ShapeDtypeStruct((B * N, NCH), jnp.float32),
    )(xyz_flat, xyzt)

    RB2 = 512
    chunkid = pl.pallas_call(
        _stage1_body,
        grid=(B * N // RB2,),
        in_specs=[pl.BlockSpec((RB2, NCH), lambda i: (i, 0))],
        out_specs=pl.BlockSpec((RB2, 16), lambda i: (i, 0)),
        out_shape=jax.ShapeDtypeStruct((B * N, 16), jnp.int32),
    )(cmin)

    ci = chunkid.reshape(B, N * 16)
    planes = [xyz[:, :, d].reshape(B, NCH, 16) for d in range(3)]
    gx, gy, gz = [
        jax.vmap(lambda p, c: p[c])(pl_, ci).reshape(B * N, 256)
        for pl_ in planes
    ]
    gidx = (jnp.repeat(chunkid, 16, axis=1) * 16
            + jnp.tile(jnp.arange(16, dtype=jnp.int32), 16)[None, :]
            ).astype(jnp.float32)

    idx16 = pl.pallas_call(
        _stage2_body,
        grid=(B * N // RB2,),
        in_specs=[
            pl.BlockSpec((RB2, 256), lambda i: (i, 0)),
            pl.BlockSpec((RB2, 256), lambda i: (i, 0)),
            pl.BlockSpec((RB2, 256), lambda i: (i, 0)),
            pl.BlockSpec((RB2, 256), lambda i: (i, 0)),
            pl.BlockSpec((RB2, 3), lambda i: (i, 0)),
        ],
        out_specs=pl.BlockSpec((RB2, 16), lambda i: (i, 0)),
        out_shape=jax.ShapeDtypeStruct((B * N, 16), jnp.int32),
    )(gx, gy, gz, gidx, xyz_flat)

    RBM = 1024
    proj = pl.pallas_call(
        _proj_body,
        grid=(B * N // RBM,),
        in_specs=[
            pl.BlockSpec((RBM, Cin), lambda i: (i, 0)),
            pl.BlockSpec((Cin, Cout), lambda i: (0, 0)),
        ],
        out_specs=pl.BlockSpec((RBM, Cout), lambda i: (i, 0)),
        out_shape=jax.ShapeDtypeStruct((B * N, Cout), jnp.float32),
    )(feature.reshape(B * N, Cin), W)

    boff = (jnp.arange(B, dtype=jnp.int32) * N)[:, None]
    flat_idx = (idx16.reshape(B, N * K) + boff).reshape(-1)
    counts = jnp.zeros((B * N,), jnp.float32).at[flat_idx].add(1.0)
    M = B * N * K
    s1 = jnp.sum(counts[:, None] * proj, axis=0)
    s2 = jnp.sum(counts[:, None] * (proj * proj), axis=0)
    mean = s1 / M
    var = s2 / M - mean * mean
    scale = gamma * jax.lax.rsqrt(var + 1e-5)
    shift = beta - mean * scale

    g = proj[flat_idx].reshape(B * N, K, Cout)
    pmax = jnp.max(g, axis=1)
    pmin = jnp.min(g, axis=1)

    out = pl.pallas_call(
        _combine_body,
        grid=(B * N // RBM,),
        in_specs=[
            pl.BlockSpec((RBM, Cout), lambda i: (i, 0)),
            pl.BlockSpec((RBM, Cout), lambda i: (i, 0)),
            pl.BlockSpec((1, Cout), lambda i: (0, 0)),
            pl.BlockSpec((1, Cout), lambda i: (0, 0)),
        ],
        out_specs=pl.BlockSpec((RBM, Cout), lambda i: (i, 0)),
        out_shape=jax.ShapeDtypeStruct((B * N, Cout), jnp.float32),
    )(pmax, pmin, scale.reshape(1, Cout), shift.reshape(1, Cout))

    return (xyz, out.reshape(B, N, Cout))


# speedup vs baseline: 1.6175x; 1.0810x over previous
import functools

import jax
import jax.numpy as jnp
from jax import lax
from jax.experimental import pallas as pl
from jax.experimental.pallas import tpu as pltpu
from jax.experimental.pallas import tpu_sc as plsc

_NEG = 3.0e38
_NW = 32


def _make_gather_minmax(BN, C, K):
    rows_w = BN // _NW
    BLK = 8
    GB = BLK * K
    nblk = rows_w // BLK
    mesh = plsc.VectorSubcoreMesh(core_axis_name="c", subcore_axis_name="s")

    @functools.partial(
        pl.kernel,
        out_type=(jax.ShapeDtypeStruct((BN, C), jnp.float32),
                  jax.ShapeDtypeStruct((BN, C), jnp.float32)),
        mesh=mesh,
        scratch_types=[
            pltpu.VMEM((nblk, GB), jnp.int32),
            pltpu.VMEM((2, GB, C), jnp.float32),
            pltpu.VMEM((2, BLK, C), jnp.float32),
            pltpu.VMEM((2, BLK, C), jnp.float32),
            pltpu.SemaphoreType.DMA,
            pltpu.SemaphoreType.DMA,
            pltpu.SemaphoreType.DMA,
        ],
    )
    def kfn(idx_hbm, proj_hbm, pmax_hbm, pmin_hbm,
            idxv, rowsv, omaxv, ominv, sg0, sg1, sout):
        wid = lax.axis_index("s") * 2 + lax.axis_index("c")
        base = wid * rows_w
        pltpu.sync_copy(idx_hbm.at[wid], idxv)
        sg = (sg0, sg1)

        def fire(blki, par):
            pltpu.async_copy(proj_hbm.at[idxv.at[blki]], rowsv.at[par],
                             sg[par])

        fire(0, 0)

        def body(i2, car):
            for par in range(2):
                blki = i2 * 2 + par

                @pl.when(blki + 1 < nblk)
                def _fire_next():
                    fire(blki + 1, 1 - par)

                pltpu.make_async_copy(
                    proj_hbm.at[idxv.at[0]], rowsv.at[par], sg[par]).wait()

                @pl.when(blki >= 2)
                def _drain_out():
                    pltpu.make_async_copy(
                        omaxv.at[par], pmax_hbm.at[pl.ds(base, BLK)],
                        sout).wait()
                    pltpu.make_async_copy(
                        ominv.at[par], pmin_hbm.at[pl.ds(base, BLK)],
                        sout).wait()

                def cb(j, c):
                    col = j * 16
                    for r in range(BLK):
                        v = rowsv[par, r * K, pl.ds(col, 16)]
                        mx = v
                        mn = v
                        for r2 in range(1, K):
                            v = rowsv[par, r * K + r2, pl.ds(col, 16)]
                            mx = jnp.maximum(mx, v)
                            mn = jnp.minimum(mn, v)
                        omaxv[par, r, pl.ds(col, 16)] = mx
                        ominv[par, r, pl.ds(col, 16)] = mn
                    return c

                lax.fori_loop(0, C // 16, cb, 0)

                row0 = base + blki * BLK
                pltpu.async_copy(omaxv.at[par],
                                 pmax_hbm.at[pl.ds(row0, BLK)], sout)
                pltpu.async_copy(ominv.at[par],
                                 pmin_hbm.at[pl.ds(row0, BLK)], sout)
            return car

        lax.fori_loop(0, nblk // 2, body, 0)
        for par in range(2):
            pltpu.make_async_copy(
                omaxv.at[par], pmax_hbm.at[pl.ds(base, BLK)], sout).wait()
            pltpu.make_async_copy(
                ominv.at[par], pmin_hbm.at[pl.ds(base, BLK)], sout).wait()

    return kfn


def _proj_body(f_ref, w_ref, o_ref):
    o_ref[...] = jax.lax.dot_general(
        f_ref[...], w_ref[...], (((1,), (0,)), ((), ())),
        preferred_element_type=jnp.float32,
        precision=jax.lax.Precision.HIGHEST,
    )


def _combine_body(pmax_ref, pmin_ref, scale_ref, shift_ref, o_ref):
    scale = scale_ref[...]
    shift = shift_ref[...]
    g = jnp.where(scale >= 0.0, pmax_ref[...], pmin_ref[...])
    o_ref[...] = jnp.maximum(g * scale + shift, 0.0)


def _bf(x):
    return x.astype(jnp.bfloat16).astype(jnp.float32)


def _dist_like_ref(rows, cands):
    s = _bf(rows[0]) * _bf(cands[0])
    s = s + _bf(rows[1]) * _bf(cands[1])
    s = s + _bf(rows[2]) * _bf(cands[2])
    nr = rows[0] * rows[0] + rows[1] * rows[1] + rows[2] * rows[2]
    nc = cands[0] * cands[0] + cands[1] * cands[1] + cands[2] * cands[2]
    return (-2.0 * s + nr) + nc


def _cmin_body(xyz_ref, xyzt_ref, cmin_ref):
    rb = xyz_ref.shape[0]
    n = xyzt_ref.shape[2]
    rows = [xyz_ref[:, d:d + 1] for d in range(3)]
    cands = [xyzt_ref[0, d:d + 1, :] for d in range(3)]
    dist = _dist_like_ref(rows, cands)
    cmin_ref[...] = jnp.min(dist.reshape(rb, n // 16, 16), axis=-1)


def _stage1_body(cmin_ref, out_ref):
    rb, nc = cmin_ref.shape
    v = cmin_ref[...]
    iota = lax.broadcasted_iota(jnp.int32, (rb, nc), 1).astype(jnp.float32)
    for t in range(16):
        m = jnp.min(v, axis=1, keepdims=True)
        eq = v <= m
        pos = jnp.min(jnp.where(eq, iota, _NEG), axis=1, keepdims=True)
        out_ref[:, t:t + 1] = pos.astype(jnp.int32)
        v = jnp.where(iota == pos, _NEG, v)


def _stage2_body(gx_ref, gy_ref, gz_ref, gidx_ref, xyz_ref, out_ref):
    rb, nc = gx_ref.shape
    rows = [xyz_ref[:, d:d + 1] for d in range(3)]
    v = _dist_like_ref(rows, [gx_ref[...], gy_ref[...], gz_ref[...]])
    gidx = gidx_ref[...]
    for t in range(16):
        m = jnp.min(v, axis=1, keepdims=True)
        eq = v <= m
        idx = jnp.min(jnp.where(eq, gidx, _NEG), axis=1, keepdims=True)
        out_ref[:, t:t + 1] = idx.astype(jnp.int32)
        v = jnp.where(jnp.logical_and(eq, gidx == idx), _NEG, v)


def kernel(xyz, feature, npoint, W, gamma, beta):
    del npoint
    B, N, Cin = feature.shape
    Cout = W.shape[1]
    K = 16
    NCH = N // 16

    RB = 256
    xyz_flat = xyz.reshape(B * N, 3)
    xyzt = jnp.transpose(xyz, (0, 2, 1))
    cmin = pl.pallas_call(
        _cmin_body,
        grid=(B, N // RB),
        in_specs=[
            pl.BlockSpec((RB, 3), lambda b, i: (b * (N // RB) + i, 0)),
            pl.BlockSpec((1, 3, N), lambda b, i: (b, 0, 0)),
        ],
        out_specs=pl.BlockSpec((RB, NCH), lambda b, i: (b * (N // RB) + i, 0)),
        out_shape=jax.ShapeDtypeStruct((B * N, NCH), jnp.float32),
    )(xyz_flat, xyzt)

    RB2 = 512
    chunkid = pl.pallas_call(
        _stage1_body,
        grid=(B * N // RB2,),
        in_specs=[pl.BlockSpec((RB2, NCH), lambda i: (i, 0))],
        out_specs=pl.BlockSpec((RB2, 16), lambda i: (i, 0)),
        out_shape=jax.ShapeDtypeStruct((B * N, 16), jnp.int32),
    )(cmin)

    ci = chunkid.reshape(B, N * 16)
    planes = [xyz[:, :, d].reshape(B, NCH, 16) for d in range(3)]
    gx, gy, gz = [
        jax.vmap(lambda p, c: p[c])(pl_, ci).reshape(B * N, 256)
        for pl_ in planes
    ]
    gidx = (jnp.repeat(chunkid, 16, axis=1) * 16
            + jnp.tile(jnp.arange(16, dtype=jnp.int32), 16)[None, :]
            ).astype(jnp.float32)

    idx16 = pl.pallas_call(
        _stage2_body,
        grid=(B * N // RB2,),
        in_specs=[
            pl.BlockSpec((RB2, 256), lambda i: (i, 0)),
            pl.BlockSpec((RB2, 256), lambda i: (i, 0)),
            pl.BlockSpec((RB2, 256), lambda i: (i, 0)),
            pl.BlockSpec((RB2, 256), lambda i: (i, 0)),
            pl.BlockSpec((RB2, 3), lambda i: (i, 0)),
        ],
        out_specs=pl.BlockSpec((RB2, 16), lambda i: (i, 0)),
        out_shape=jax.ShapeDtypeStruct((B * N, 16), jnp.int32),
    )(gx, gy, gz, gidx, xyz_flat)

    RBM = 1024
    proj = pl.pallas_call(
        _proj_body,
        grid=(B * N // RBM,),
        in_specs=[
            pl.BlockSpec((RBM, Cin), lambda i: (i, 0)),
            pl.BlockSpec((Cin, Cout), lambda i: (0, 0)),
        ],
        out_specs=pl.BlockSpec((RBM, Cout), lambda i: (i, 0)),
        out_shape=jax.ShapeDtypeStruct((B * N, Cout), jnp.float32),
    )(feature.reshape(B * N, Cin), W)

    boff = (jnp.arange(B, dtype=jnp.int32) * N)[:, None]
    flat_idx = (idx16.reshape(B, N * K) + boff).reshape(-1)
    counts = jnp.zeros((B * N,), jnp.float32).at[flat_idx].add(1.0)
    M = B * N * K
    s1 = jnp.sum(counts[:, None] * proj, axis=0)
    s2 = jnp.sum(counts[:, None] * (proj * proj), axis=0)
    mean = s1 / M
    var = s2 / M - mean * mean
    scale = gamma * jax.lax.rsqrt(var + 1e-5)
    shift = beta - mean * scale

    BN = B * N
    idx_sc = flat_idx.reshape(_NW, BN * K // (_NW * 128), 128)
    pmax, pmin = _make_gather_minmax(BN, Cout, K)(idx_sc, proj)

    out = pl.pallas_call(
        _combine_body,
        grid=(B * N // RBM,),
        in_specs=[
            pl.BlockSpec((RBM, Cout), lambda i: (i, 0)),
            pl.BlockSpec((RBM, Cout), lambda i: (i, 0)),
            pl.BlockSpec((1, Cout), lambda i: (0, 0)),
            pl.BlockSpec((1, Cout), lambda i: (0, 0)),
        ],
        out_specs=pl.BlockSpec((RBM, Cout), lambda i: (i, 0)),
        out_shape=jax.ShapeDtypeStruct((B * N, Cout), jnp.float32),
    )(pmax, pmin, scale.reshape(1, Cout), shift.reshape(1, Cout))

    return (xyz, out.reshape(B, N, Cout))


# speedup vs baseline: 8.7230x; 5.3930x over previous
import functools

import jax
import jax.numpy as jnp
from jax import lax
from jax.experimental import pallas as pl
from jax.experimental.pallas import tpu as pltpu
from jax.experimental.pallas import tpu_sc as plsc

_NEG = 3.0e38
_NW = 32


def _make_gather_minmax(BN, C, K):
    rows_w = BN // _NW
    BLK = 4
    GB = BLK * K
    nblk = rows_w // BLK
    mesh = plsc.VectorSubcoreMesh(core_axis_name="c", subcore_axis_name="s")

    @functools.partial(
        pl.kernel,
        out_type=(jax.ShapeDtypeStruct((BN, C), jnp.float32),
                  jax.ShapeDtypeStruct((BN, C), jnp.float32)),
        mesh=mesh,
        scratch_types=[
            pltpu.VMEM((nblk, GB), jnp.int32),
            pltpu.VMEM((2, GB, C), jnp.float32),
            pltpu.VMEM((2, BLK, C), jnp.float32),
            pltpu.VMEM((2, BLK, C), jnp.float32),
            pltpu.SemaphoreType.DMA,
            pltpu.SemaphoreType.DMA,
            pltpu.SemaphoreType.DMA,
        ],
    )
    def kfn(idx_hbm, proj_hbm, pmax_hbm, pmin_hbm,
            idxv, rowsv, omaxv, ominv, sg0, sg1, sout):
        wid = lax.axis_index("s") * 2 + lax.axis_index("c")
        base = wid * rows_w
        pltpu.sync_copy(idx_hbm.at[wid], idxv)
        sg = (sg0, sg1)

        def fire(blki, par):
            pltpu.async_copy(proj_hbm.at[idxv.at[blki]], rowsv.at[par],
                             sg[par])

        fire(0, 0)

        def body(i2, car):
            for par in range(2):
                blki = i2 * 2 + par

                @pl.when(blki + 1 < nblk)
                def _fire_next():
                    fire(blki + 1, 1 - par)

                pltpu.make_async_copy(
                    proj_hbm.at[idxv.at[0]], rowsv.at[par], sg[par]).wait()

                @pl.when(blki >= 2)
                def _drain_out():
                    pltpu.make_async_copy(
                        omaxv.at[par], pmax_hbm.at[pl.ds(base, BLK)],
                        sout).wait()
                    pltpu.make_async_copy(
                        ominv.at[par], pmin_hbm.at[pl.ds(base, BLK)],
                        sout).wait()

                def cb(j, c):
                    col = j * 16
                    for r in range(BLK):
                        v = rowsv[par, r * K, pl.ds(col, 16)]
                        mx = v
                        mn = v
                        for r2 in range(1, K):
                            v = rowsv[par, r * K + r2, pl.ds(col, 16)]
                            mx = jnp.maximum(mx, v)
                            mn = jnp.minimum(mn, v)
                        omaxv[par, r, pl.ds(col, 16)] = mx
                        ominv[par, r, pl.ds(col, 16)] = mn
                    return c

                lax.fori_loop(0, C // 16, cb, 0)

                row0 = base + blki * BLK
                pltpu.async_copy(omaxv.at[par],
                                 pmax_hbm.at[pl.ds(row0, BLK)], sout)
                pltpu.async_copy(ominv.at[par],
                                 pmin_hbm.at[pl.ds(row0, BLK)], sout)
            return car

        lax.fori_loop(0, nblk // 2, body, 0)
        for par in range(2):
            pltpu.make_async_copy(
                omaxv.at[par], pmax_hbm.at[pl.ds(base, BLK)], sout).wait()
            pltpu.make_async_copy(
                ominv.at[par], pmin_hbm.at[pl.ds(base, BLK)], sout).wait()

    return kfn


def _make_gather_chunks(BN, K):
    rows_w = BN // _NW
    BLK = 4
    GB = BLK * K
    nblk = rows_w // BLK
    mesh = plsc.VectorSubcoreMesh(core_axis_name="c", subcore_axis_name="s")
    out_t = jax.ShapeDtypeStruct((BN * K, 16), jnp.float32)

    @functools.partial(
        pl.kernel,
        out_type=(out_t, out_t, out_t),
        mesh=mesh,
        scratch_types=[
            pltpu.VMEM((nblk, GB), jnp.int32),
            pltpu.VMEM((2, GB, 128), jnp.float32),
            pltpu.VMEM((2, GB, 16), jnp.float32),
            pltpu.VMEM((2, GB, 16), jnp.float32),
            pltpu.VMEM((2, GB, 16), jnp.float32),
            pltpu.SemaphoreType.DMA,
            pltpu.SemaphoreType.DMA,
        ],
    )
    def kfn(idx_hbm, tab_hbm, gx_hbm, gy_hbm, gz_hbm,
            idxv, gbuf, bx, by, bz, sg0, sg1):
        wid = lax.axis_index("s") * 2 + lax.axis_index("c")
        base = wid * rows_w
        pltpu.sync_copy(idx_hbm.at[wid], idxv)
        sg = (sg0, sg1)

        def fire(blki, par):
            pltpu.async_copy(tab_hbm.at[idxv.at[blki]], gbuf.at[par],
                             sg[par])

        fire(0, 0)
        fire(1, 1)

        def body(i2, car):
            for par in range(2):
                blki = i2 * 2 + par
                pltpu.make_async_copy(
                    tab_hbm.at[idxv.at[0]], gbuf.at[par], sg[par]).wait()

                def cb(rr, c):
                    bx[par, rr, :] = gbuf[par, rr, pl.ds(0, 16)]
                    by[par, rr, :] = gbuf[par, rr, pl.ds(16, 16)]
                    bz[par, rr, :] = gbuf[par, rr, pl.ds(32, 16)]
                    return c

                lax.fori_loop(0, GB, cb, 0)
                row0 = (base + blki * BLK) * K
                for o, bf in zip((gx_hbm, gy_hbm, gz_hbm), (bx, by, bz)):
                    pltpu.sync_copy(bf.at[par], o.at[pl.ds(row0, GB)])

                @pl.when(blki + 2 < nblk)
                def _fire_next():
                    fire(blki + 2, par)
            return car

        lax.fori_loop(0, nblk // 2, body, 0)

    return kfn


def _proj_body(f_ref, w_ref, o_ref):
    o_ref[...] = jax.lax.dot_general(
        f_ref[...], w_ref[...], (((1,), (0,)), ((), ())),
        preferred_element_type=jnp.float32,
        precision=jax.lax.Precision.HIGHEST,
    )


def _combine_body(pmax_ref, pmin_ref, scale_ref, shift_ref, o_ref):
    scale = scale_ref[...]
    shift = shift_ref[...]
    g = jnp.where(scale >= 0.0, pmax_ref[...], pmin_ref[...])
    o_ref[...] = jnp.maximum(g * scale + shift, 0.0)


def _bf(x):
    return x.astype(jnp.bfloat16).astype(jnp.float32)


def _dist_like_ref(rows, cands):
    s = _bf(rows[0]) * _bf(cands[0])
    s = s + _bf(rows[1]) * _bf(cands[1])
    s = s + _bf(rows[2]) * _bf(cands[2])
    nr = rows[0] * rows[0] + rows[1] * rows[1] + rows[2] * rows[2]
    nc = cands[0] * cands[0] + cands[1] * cands[1] + cands[2] * cands[2]
    return (-2.0 * s + nr) + nc


def _cmin_body(xyz_ref, xyzt_ref, cmin_ref):
    rb = xyz_ref.shape[0]
    n = xyzt_ref.shape[2]
    rows = [xyz_ref[:, d:d + 1] for d in range(3)]
    cands = [xyzt_ref[0, d:d + 1, :] for d in range(3)]
    dist = _dist_like_ref(rows, cands)
    cmin_ref[...] = jnp.min(dist.reshape(rb, n // 16, 16), axis=-1)


def _stage1_body(cmin_ref, out_ref):
    rb, nc = cmin_ref.shape
    v = cmin_ref[...]
    iota = lax.broadcasted_iota(jnp.int32, (rb, nc), 1).astype(jnp.float32)
    for t in range(16):
        m = jnp.min(v, axis=1, keepdims=True)
        eq = v <= m
        pos = jnp.min(jnp.where(eq, iota, _NEG), axis=1, keepdims=True)
        out_ref[:, t:t + 1] = pos.astype(jnp.int32)
        v = jnp.where(iota == pos, _NEG, v)


def _stage2_body(gx_ref, gy_ref, gz_ref, gidx_ref, xyz_ref, out_ref):
    rb, nc = gx_ref.shape
    rows = [xyz_ref[:, d:d + 1] for d in range(3)]
    v = _dist_like_ref(rows, [gx_ref[...], gy_ref[...], gz_ref[...]])
    gidx = gidx_ref[...]
    for t in range(16):
        m = jnp.min(v, axis=1, keepdims=True)
        eq = v <= m
        idx = jnp.min(jnp.where(eq, gidx, _NEG), axis=1, keepdims=True)
        out_ref[:, t:t + 1] = idx.astype(jnp.int32)
        v = jnp.where(jnp.logical_and(eq, gidx == idx), _NEG, v)


def kernel(xyz, feature, npoint, W, gamma, beta):
    del npoint
    B, N, Cin = feature.shape
    Cout = W.shape[1]
    K = 16
    NCH = N // 16

    RB = 256
    xyz_flat = xyz.reshape(B * N, 3)
    xyzt = jnp.transpose(xyz, (0, 2, 1))
    cmin = pl.pallas_call(
        _cmin_body,
        grid=(B, N // RB),
        in_specs=[
            pl.BlockSpec((RB, 3), lambda b, i: (b * (N // RB) + i, 0)),
            pl.BlockSpec((1, 3, N), lambda b, i: (b, 0, 0)),
        ],
        out_specs=pl.BlockSpec((RB, NCH), lambda b, i: (b * (N // RB) + i, 0)),
        out_shape=jax.ShapeDtypeStruct((B * N, NCH), jnp.float32),
    )(xyz_flat, xyzt)

    RB2 = 512
    chunkid = pl.pallas_call(
        _stage1_body,
        grid=(B * N // RB2,),
        in_specs=[pl.BlockSpec((RB2, NCH), lambda i: (i, 0))],
        out_specs=pl.BlockSpec((RB2, 16), lambda i: (i, 0)),
        out_shape=jax.ShapeDtypeStruct((B * N, 16), jnp.int32),
    )(cmin)

    BN = B * N
    boff2 = (jnp.arange(B, dtype=jnp.int32) * NCH)[:, None]
    cid_g = (chunkid.reshape(B, N * 16) + boff2).reshape(_NW, BN * 16 // (_NW * 64), 64)
    tab = jnp.pad(
        jnp.transpose(xyz.reshape(B * NCH, 16, 3), (0, 2, 1)).reshape(
            B * NCH, 48),
        ((0, 0), (0, 80)))
    gxf, gyf, gzf = _make_gather_chunks(BN, 16)(cid_g, tab)
    gx = gxf.reshape(BN, 256)
    gy = gyf.reshape(BN, 256)
    gz = gzf.reshape(BN, 256)
    gidx = (jnp.repeat(chunkid, 16, axis=1) * 16
            + jnp.tile(jnp.arange(16, dtype=jnp.int32), 16)[None, :]
            ).astype(jnp.float32)

    idx16 = pl.pallas_call(
        _stage2_body,
        grid=(B * N // RB2,),
        in_specs=[
            pl.BlockSpec((RB2, 256), lambda i: (i, 0)),
            pl.BlockSpec((RB2, 256), lambda i: (i, 0)),
            pl.BlockSpec((RB2, 256), lambda i: (i, 0)),
            pl.BlockSpec((RB2, 256), lambda i: (i, 0)),
            pl.BlockSpec((RB2, 3), lambda i: (i, 0)),
        ],
        out_specs=pl.BlockSpec((RB2, 16), lambda i: (i, 0)),
        out_shape=jax.ShapeDtypeStruct((B * N, 16), jnp.int32),
    )(gx, gy, gz, gidx, xyz_flat)

    RBM = 1024
    proj = pl.pallas_call(
        _proj_body,
        grid=(B * N // RBM,),
        in_specs=[
            pl.BlockSpec((RBM, Cin), lambda i: (i, 0)),
            pl.BlockSpec((Cin, Cout), lambda i: (0, 0)),
        ],
        out_specs=pl.BlockSpec((RBM, Cout), lambda i: (i, 0)),
        out_shape=jax.ShapeDtypeStruct((B * N, Cout), jnp.float32),
    )(feature.reshape(B * N, Cin), W)

    boff = (jnp.arange(B, dtype=jnp.int32) * N)[:, None]
    flat_idx = (idx16.reshape(B, N * K) + boff).reshape(-1)
    counts = jnp.zeros((B * N,), jnp.float32).at[flat_idx].add(1.0)
    M = B * N * K
    s1 = jnp.sum(counts[:, None] * proj, axis=0)
    s2 = jnp.sum(counts[:, None] * (proj * proj), axis=0)
    mean = s1 / M
    var = s2 / M - mean * mean
    scale = gamma * jax.lax.rsqrt(var + 1e-5)
    shift = beta - mean * scale

    idx_sc = flat_idx.reshape(_NW, BN * K // (_NW * 64), 64)
    pmax, pmin = _make_gather_minmax(BN, Cout, K)(idx_sc, proj)

    out = pl.pallas_call(
        _combine_body,
        grid=(B * N // RBM,),
        in_specs=[
            pl.BlockSpec((RBM, Cout), lambda i: (i, 0)),
            pl.BlockSpec((RBM, Cout), lambda i: (i, 0)),
            pl.BlockSpec((1, Cout), lambda i: (0, 0)),
            pl.BlockSpec((1, Cout), lambda i: (0, 0)),
        ],
        out_specs=pl.BlockSpec((RBM, Cout), lambda i: (i, 0)),
        out_shape=jax.ShapeDtypeStruct((B * N, Cout), jnp.float32),
    )(pmax, pmin, scale.reshape(1, Cout), shift.reshape(1, Cout))

    return (xyz, out.reshape(B, N, Cout))


# speedup vs baseline: 15.4718x; 1.7737x over previous
import functools

import jax
import jax.numpy as jnp
from jax import lax
from jax.experimental import pallas as pl
from jax.experimental.pallas import tpu as pltpu
from jax.experimental.pallas import tpu_sc as plsc

_NEG = 3.0e38
_NW = 32


def _make_gather_minmax(BN, C, K):
    rows_w = BN // _NW
    BLK = 4
    GB = BLK * K
    nblk = rows_w // BLK
    mesh = plsc.VectorSubcoreMesh(core_axis_name="c", subcore_axis_name="s")

    @functools.partial(
        pl.kernel,
        out_type=(jax.ShapeDtypeStruct((BN, C), jnp.float32),
                  jax.ShapeDtypeStruct((BN, C), jnp.float32)),
        mesh=mesh,
        scratch_types=[
            pltpu.VMEM((nblk, GB), jnp.int32),
            pltpu.VMEM((2, GB, C), jnp.float32),
            pltpu.VMEM((2, BLK, C), jnp.float32),
            pltpu.VMEM((2, BLK, C), jnp.float32),
            pltpu.SemaphoreType.DMA,
            pltpu.SemaphoreType.DMA,
            pltpu.SemaphoreType.DMA,
        ],
    )
    def kfn(idx_hbm, proj_hbm, pmax_hbm, pmin_hbm,
            idxv, rowsv, omaxv, ominv, sg0, sg1, sout):
        wid = lax.axis_index("s") * 2 + lax.axis_index("c")
        base = wid * rows_w
        pltpu.sync_copy(idx_hbm.at[wid], idxv)
        sg = (sg0, sg1)

        def fire(blki, par):
            pltpu.async_copy(proj_hbm.at[idxv.at[blki]], rowsv.at[par],
                             sg[par])

        fire(0, 0)

        def body(i2, car):
            for par in range(2):
                blki = i2 * 2 + par

                @pl.when(blki + 1 < nblk)
                def _fire_next():
                    fire(blki + 1, 1 - par)

                pltpu.make_async_copy(
                    proj_hbm.at[idxv.at[0]], rowsv.at[par], sg[par]).wait()

                @pl.when(blki >= 2)
                def _drain_out():
                    pltpu.make_async_copy(
                        omaxv.at[par], pmax_hbm.at[pl.ds(base, BLK)],
                        sout).wait()
                    pltpu.make_async_copy(
                        ominv.at[par], pmin_hbm.at[pl.ds(base, BLK)],
                        sout).wait()

                def cb(j, c):
                    col = j * 16
                    for r in range(BLK):
                        v = rowsv[par, r * K, pl.ds(col, 16)]
                        mx = v
                        mn = v
                        for r2 in range(1, K):
                            v = rowsv[par, r * K + r2, pl.ds(col, 16)]
                            mx = jnp.maximum(mx, v)
                            mn = jnp.minimum(mn, v)
                        omaxv[par, r, pl.ds(col, 16)] = mx
                        ominv[par, r, pl.ds(col, 16)] = mn
                    return c

                lax.fori_loop(0, C // 16, cb, 0)

                row0 = base + blki * BLK
                pltpu.async_copy(omaxv.at[par],
                                 pmax_hbm.at[pl.ds(row0, BLK)], sout)
                pltpu.async_copy(ominv.at[par],
                                 pmin_hbm.at[pl.ds(row0, BLK)], sout)
            return car

        lax.fori_loop(0, nblk // 2, body, 0)
        for par in range(2):
            pltpu.make_async_copy(
                omaxv.at[par], pmax_hbm.at[pl.ds(base, BLK)], sout).wait()
            pltpu.make_async_copy(
                ominv.at[par], pmin_hbm.at[pl.ds(base, BLK)], sout).wait()

    return kfn


def _make_gather_chunks(BN, K):
    rows_w = BN // _NW
    BLK = 4
    GB = BLK * K
    nblk = rows_w // BLK
    mesh = plsc.VectorSubcoreMesh(core_axis_name="c", subcore_axis_name="s")
    out_t = jax.ShapeDtypeStruct((BN * K, 16), jnp.float32)

    @functools.partial(
        pl.kernel,
        out_type=(out_t, out_t, out_t),
        mesh=mesh,
        scratch_types=[
            pltpu.VMEM((nblk, GB), jnp.int32),
            pltpu.VMEM((2, GB, 128), jnp.float32),
            pltpu.VMEM((2, GB, 16), jnp.float32),
            pltpu.VMEM((2, GB, 16), jnp.float32),
            pltpu.VMEM((2, GB, 16), jnp.float32),
            pltpu.SemaphoreType.DMA,
            pltpu.SemaphoreType.DMA,
        ],
    )
    def kfn(idx_hbm, tab_hbm, gx_hbm, gy_hbm, gz_hbm,
            idxv, gbuf, bx, by, bz, sg0, sg1):
        wid = lax.axis_index("s") * 2 + lax.axis_index("c")
        base = wid * rows_w
        pltpu.sync_copy(idx_hbm.at[wid], idxv)
        sg = (sg0, sg1)

        def fire(blki, par):
            pltpu.async_copy(tab_hbm.at[idxv.at[blki]], gbuf.at[par],
                             sg[par])

        fire(0, 0)
        fire(1, 1)

        def body(i2, car):
            for par in range(2):
                blki = i2 * 2 + par
                pltpu.make_async_copy(
                    tab_hbm.at[idxv.at[0]], gbuf.at[par], sg[par]).wait()

                def cb(rr, c):
                    bx[par, rr, :] = gbuf[par, rr, pl.ds(0, 16)]
                    by[par, rr, :] = gbuf[par, rr, pl.ds(16, 16)]
                    bz[par, rr, :] = gbuf[par, rr, pl.ds(32, 16)]
                    return c

                lax.fori_loop(0, GB, cb, 0)
                row0 = (base + blki * BLK) * K
                for o, bf in zip((gx_hbm, gy_hbm, gz_hbm), (bx, by, bz)):
                    pltpu.sync_copy(bf.at[par], o.at[pl.ds(row0, GB)])

                @pl.when(blki + 2 < nblk)
                def _fire_next():
                    fire(blki + 2, par)
            return car

        lax.fori_loop(0, nblk // 2, body, 0)

    return kfn


def _proj_body(f_ref, w_ref, o_ref):
    o_ref[...] = jax.lax.dot_general(
        f_ref[...], w_ref[...], (((1,), (0,)), ((), ())),
        preferred_element_type=jnp.float32,
        precision=jax.lax.Precision.HIGHEST,
    )


def _combine_body(pmax_ref, pmin_ref, scale_ref, shift_ref, o_ref):
    scale = scale_ref[...]
    shift = shift_ref[...]
    g = jnp.where(scale >= 0.0, pmax_ref[...], pmin_ref[...])
    o_ref[...] = jnp.maximum(g * scale + shift, 0.0)


def _bf(x):
    return x.astype(jnp.bfloat16).astype(jnp.float32)


def _dist_like_ref(rows, cands):
    s = _bf(rows[0]) * _bf(cands[0])
    s = s + _bf(rows[1]) * _bf(cands[1])
    s = s + _bf(rows[2]) * _bf(cands[2])
    nr = rows[0] * rows[0] + rows[1] * rows[1] + rows[2] * rows[2]
    nc = cands[0] * cands[0] + cands[1] * cands[1] + cands[2] * cands[2]
    return (-2.0 * s + nr) + nc


def _cmin_body(xyz_ref, xyzt_ref, cmin_ref):
    rows = [xyz_ref[:, d:d + 1] for d in range(3)]
    m = None
    for o in range(16):
        cands = [xyzt_ref[0, d, o:o + 1, :] for d in range(3)]
        dist = _dist_like_ref(rows, cands)
        m = dist if m is None else jnp.minimum(m, dist)
    cmin_ref[...] = m


def _stage1_body(cmin_ref, out_ref):
    rb, nc = cmin_ref.shape
    v = cmin_ref[...]
    iota = lax.broadcasted_iota(jnp.int32, (rb, nc), 1).astype(jnp.float32)
    for t in range(16):
        m = jnp.min(v, axis=1, keepdims=True)
        eq = v <= m
        pos = jnp.min(jnp.where(eq, iota, _NEG), axis=1, keepdims=True)
        out_ref[:, t:t + 1] = pos.astype(jnp.int32)
        v = jnp.where(iota == pos, _NEG, v)


def _stage2_body(gx_ref, gy_ref, gz_ref, gidx_ref, xyz_ref, out_ref):
    rb, nc = gx_ref.shape
    rows = [xyz_ref[:, d:d + 1] for d in range(3)]
    v = _dist_like_ref(rows, [gx_ref[...], gy_ref[...], gz_ref[...]])
    gidx = gidx_ref[...]
    for t in range(16):
        m = jnp.min(v, axis=1, keepdims=True)
        eq = v <= m
        idx = jnp.min(jnp.where(eq, gidx, _NEG), axis=1, keepdims=True)
        out_ref[:, t:t + 1] = idx.astype(jnp.int32)
        v = jnp.where(jnp.logical_and(eq, gidx == idx), _NEG, v)


def kernel(xyz, feature, npoint, W, gamma, beta):
    del npoint
    B, N, Cin = feature.shape
    Cout = W.shape[1]
    K = 16
    NCH = N // 16

    RB = 256
    xyz_flat = xyz.reshape(B * N, 3)
    xyzt = jnp.transpose(xyz.reshape(B, NCH, 16, 3), (0, 3, 2, 1))
    cmin = pl.pallas_call(
        _cmin_body,
        grid=(B, N // RB),
        in_specs=[
            pl.BlockSpec((RB, 3), lambda b, i: (b * (N // RB) + i, 0)),
            pl.BlockSpec((1, 3, 16, NCH), lambda b, i: (b, 0, 0, 0)),
        ],
        out_specs=pl.BlockSpec((RB, NCH), lambda b, i: (b * (N // RB) + i, 0)),
        out_shape=jax.ShapeDtypeStruct((B * N, NCH), jnp.float32),
    )(xyz_flat, xyzt)

    RB2 = 512
    chunkid = pl.pallas_call(
        _stage1_body,
        grid=(B * N // RB2,),
        in_specs=[pl.BlockSpec((RB2, NCH), lambda i: (i, 0))],
        out_specs=pl.BlockSpec((RB2, 16), lambda i: (i, 0)),
        out_shape=jax.ShapeDtypeStruct((B * N, 16), jnp.int32),
    )(cmin)

    BN = B * N
    boff2 = (jnp.arange(B, dtype=jnp.int32) * NCH)[:, None]
    cid_g = (chunkid.reshape(B, N * 16) + boff2).reshape(_NW, BN * 16 // (_NW * 64), 64)
    tab = jnp.pad(
        jnp.transpose(xyz.reshape(B * NCH, 16, 3), (0, 2, 1)).reshape(
            B * NCH, 48),
        ((0, 0), (0, 80)))
    gxf, gyf, gzf = _make_gather_chunks(BN, 16)(cid_g, tab)
    gx = gxf.reshape(BN, 256)
    gy = gyf.reshape(BN, 256)
    gz = gzf.reshape(BN, 256)
    gidx = (jnp.repeat(chunkid, 16, axis=1) * 16
            + jnp.tile(jnp.arange(16, dtype=jnp.int32), 16)[None, :]
            ).astype(jnp.float32)

    idx16 = pl.pallas_call(
        _stage2_body,
        grid=(B * N // RB2,),
        in_specs=[
            pl.BlockSpec((RB2, 256), lambda i: (i, 0)),
            pl.BlockSpec((RB2, 256), lambda i: (i, 0)),
            pl.BlockSpec((RB2, 256), lambda i: (i, 0)),
            pl.BlockSpec((RB2, 256), lambda i: (i, 0)),
            pl.BlockSpec((RB2, 3), lambda i: (i, 0)),
        ],
        out_specs=pl.BlockSpec((RB2, 16), lambda i: (i, 0)),
        out_shape=jax.ShapeDtypeStruct((B * N, 16), jnp.int32),
    )(gx, gy, gz, gidx, xyz_flat)

    RBM = 1024
    proj = pl.pallas_call(
        _proj_body,
        grid=(B * N // RBM,),
        in_specs=[
            pl.BlockSpec((RBM, Cin), lambda i: (i, 0)),
            pl.BlockSpec((Cin, Cout), lambda i: (0, 0)),
        ],
        out_specs=pl.BlockSpec((RBM, Cout), lambda i: (i, 0)),
        out_shape=jax.ShapeDtypeStruct((B * N, Cout), jnp.float32),
    )(feature.reshape(B * N, Cin), W)

    boff = (jnp.arange(B, dtype=jnp.int32) * N)[:, None]
    flat_idx = (idx16.reshape(B, N * K) + boff).reshape(-1)
    counts = jnp.zeros((B * N,), jnp.float32).at[flat_idx].add(1.0)
    M = B * N * K
    s1 = jnp.sum(counts[:, None] * proj, axis=0)
    s2 = jnp.sum(counts[:, None] * (proj * proj), axis=0)
    mean = s1 / M
    var = s2 / M - mean * mean
    scale = gamma * jax.lax.rsqrt(var + 1e-5)
    shift = beta - mean * scale

    idx_sc = flat_idx.reshape(_NW, BN * K // (_NW * 64), 64)
    pmax, pmin = _make_gather_minmax(BN, Cout, K)(idx_sc, proj)

    out = pl.pallas_call(
        _combine_body,
        grid=(B * N // RBM,),
        in_specs=[
            pl.BlockSpec((RBM, Cout), lambda i: (i, 0)),
            pl.BlockSpec((RBM, Cout), lambda i: (i, 0)),
            pl.BlockSpec((1, Cout), lambda i: (0, 0)),
            pl.BlockSpec((1, Cout), lambda i: (0, 0)),
        ],
        out_specs=pl.BlockSpec((RBM, Cout), lambda i: (i, 0)),
        out_shape=jax.ShapeDtypeStruct((B * N, Cout), jnp.float32),
    )(pmax, pmin, scale.reshape(1, Cout), shift.reshape(1, Cout))

    return (xyz, out.reshape(B, N, Cout))
